# baseline (speedup 1.0000x reference)
import jax
import jax.numpy as jnp
from jax.experimental import pallas as pl

SIZES = [5023, 1256, 314, 79, 20]
P = [s + 1 for s in SIZES]
SP = 12
LATENT = 128
FE = [3, 16, 16, 16, 32]
FD = [32, 16, 16, 16, 3]
B = 16


def _spiral_conv(x, S, W, b, act):
    bs, npts, f = x.shape
    g = x[:, S, :].reshape(bs, npts, SP * f)
    out = act(jnp.matmul(g, W) + b)
    mask = jnp.ones((npts, 1), out.dtype).at[npts - 1, 0].set(0.0)
    return out * mask


def _copy_kernel(x_ref, o_ref):
    o_ref[...] = x_ref[...]


def kernel(x, s0, s1, s2, s3, D0, D1, D2, D3, U0, U1, U2, U3, We0, be0, We1, be1, We2, be2, We3, be3, Wfe, bfe, Wfd, bfd, Wd0, bd0, Wd1, bd1, Wd2, bd2, Wd3, bd3):
    S = [s0, s1, s2, s3]
    D = [D0, D1, D2, D3]
    U = [U0, U1, U2, U3]
    We = [We0, We1, We2, We3]
    be = [be0, be1, be2, be3]
    Wd = [Wd0, Wd1, Wd2, Wd3]
    bd = [bd0, bd1, bd2, bd3]
    elu = jax.nn.elu
    ident = lambda t: t
    h = x
    for i in range(4):
        h = _spiral_conv(h, S[i], We[i], be[i], elu)
        h = jnp.matmul(D[i], h)
    z = jnp.matmul(h.reshape(h.shape[0], -1), Wfe) + bfe
    y = (jnp.matmul(z, Wfd) + bfd).reshape(z.shape[0], P[4], FD[0])
    for i in range(4):
        y = jnp.matmul(U[3 - i], y)
        act = elu if i < 3 else ident
        y = _spiral_conv(y, S[3 - i], Wd[i], bd[i], act)
    yf = y.reshape(-1, 128)
    yf = pl.pallas_call(
        _copy_kernel,
        out_shape=jax.ShapeDtypeStruct(yf.shape, yf.dtype),
    )(yf)
    return yf.reshape(y.shape)


# speedup vs baseline: 1.2073x; 1.2073x over previous
import functools

import jax
import jax.numpy as jnp
from jax import lax
from jax.experimental import pallas as pl
from jax.experimental.pallas import tpu as pltpu
from jax.experimental.pallas import tpu_sc as plsc

_P = [5024, 1257, 315, 80, 21]
_PP = [5120, 1280, 320, 128, 32]
_SP = 12
_B = 16
_FE = [3, 16, 16, 16, 32]
_FD = [32, 16, 16, 16, 3]
_LATENT = 128
_NW = 32


def _sc_gather(table, idx_pad):
    n_pad = idx_pad.shape[0]
    d = table.shape[1]
    n_per_w = n_pad // _NW
    c = n_per_w
    while c * d * 4 > 262144:
        c //= 2
    assert c % 8 == 0 and n_per_w % c == 0
    nchunk = n_per_w // c
    mesh = plsc.VectorSubcoreMesh(core_axis_name="c", subcore_axis_name="s")

    @functools.partial(
        pl.kernel,
        out_type=jax.ShapeDtypeStruct((n_pad, d), jnp.float32),
        mesh=mesh,
        scratch_types=[
            pltpu.VMEM((c,), jnp.int32),
            pltpu.VMEM((c, d), jnp.float32),
            pltpu.SemaphoreType.DMA,
        ],
    )
    def k(table_hbm, idx_hbm, out_hbm, idx_v, rows_v, sem):
        wid = lax.axis_index("s") * 2 + lax.axis_index("c")
        base = wid * n_per_w
        for g in range(nchunk):
            off = base + g * c
            pltpu.sync_copy(idx_hbm.at[pl.ds(off, c)], idx_v)
            pltpu.async_copy(table_hbm.at[idx_v], rows_v, sem).wait()
            pltpu.sync_copy(rows_v, out_hbm.at[pl.ds(off, c)])

    return k(table, idx_pad)


def _conv(g3, wblk, bias_row, act, final_mask_limit=None):
    pp, _, bf = g3.shape
    bfo = wblk.shape[2]
    r = 256 if pp % 256 == 0 else (pp if pp <= 256 else 160)
    grid = pp // r

    def body(g_ref, w_ref, b_ref, o_ref):
        acc = jnp.zeros((r, bfo), jnp.float32)
        for j in range(_SP):
            acc = acc + jnp.dot(g_ref[:, j, :], w_ref[j],
                                preferred_element_type=jnp.float32)
        acc = acc + b_ref[...]
        if act:
            acc = jnp.where(acc > 0, acc, jnp.exp(jnp.minimum(acc, 0.0)) - 1.0)
        if final_mask_limit is not None:
            rowid = lax.broadcasted_iota(jnp.int32, (r, 1), 0) + pl.program_id(0) * r
            acc = jnp.where(rowid < final_mask_limit, acc, 0.0)
        o_ref[...] = acc

    return pl.pallas_call(
        body,
        grid=(grid,),
        in_specs=[
            pl.BlockSpec((r, _SP, bf), lambda i: (i, 0, 0)),
            pl.BlockSpec((_SP, bf, bfo), lambda i: (0, 0, 0)),
            pl.BlockSpec((1, bfo), lambda i: (0, 0)),
        ],
        out_specs=pl.BlockSpec((r, bfo), lambda i: (i, 0)),
        out_shape=jax.ShapeDtypeStruct((pp, bfo), jnp.float32),
    )(g3, wblk, bias_row)


def _pool(a, t, mask_limit):
    m, kk = a.shape
    kp, n = t.shape
    mb = min(256, m)
    kb = min(512, -(-kk // 128) * 128)
    gm = -(-m // mb)
    gk = -(-kk // kb)

    def body(a_ref, t_ref, o_ref):
        k = pl.program_id(1)
        ablk = a_ref[...]
        colid = lax.broadcasted_iota(jnp.int32, (mb, kb), 1) + k * kb
        ablk = jnp.where(colid < mask_limit, ablk, 0.0)
        tblk = t_ref[...]
        rowid = lax.broadcasted_iota(jnp.int32, (kb, n), 0) + k * kb
        tblk = jnp.where(rowid < mask_limit, tblk, 0.0)

        @pl.when(k == 0)
        def _():
            o_ref[...] = jnp.zeros_like(o_ref)

        o_ref[...] += jnp.dot(ablk, tblk, preferred_element_type=jnp.float32)

    return pl.pallas_call(
        body,
        grid=(gm, gk),
        in_specs=[
            pl.BlockSpec((mb, kb), lambda i, k: (i, k)),
            pl.BlockSpec((kb, n), lambda i, k: (k, 0)),
        ],
        out_specs=pl.BlockSpec((mb, n), lambda i, k: (i, 0)),
        out_shape=jax.ShapeDtypeStruct((m, n), jnp.float32),
    )(a, t)


def _fc(h4std, wfe, bfe_row, wfd, bfd_row):
    bsz, fin = h4std.shape
    fout = wfd.shape[1]

    def body(h_ref, a_ref, ab_ref, c_ref, cb_ref, o_ref):
        z = jnp.dot(h_ref[...], a_ref[...], preferred_element_type=jnp.float32)
        z = z + ab_ref[...]
        y = jnp.dot(z, c_ref[...], preferred_element_type=jnp.float32)
        o_ref[...] = y + cb_ref[...]

    return pl.pallas_call(
        body,
        out_shape=jax.ShapeDtypeStruct((bsz, fout), jnp.float32),
    )(h4std, wfe, bfe_row, wfd, bfd_row)


def _blockdiag(w, f_in, f_out):
    w3 = w.reshape(_SP, f_in, f_out)
    eye = jnp.eye(_B, dtype=jnp.float32)
    out = jnp.einsum('bB,jcd->jbcBd', eye, w3)
    return out.reshape(_SP, _B * f_in, _B * f_out)


def _packed_bias(b):
    return jnp.tile(b, _B)[None, :]


def _pad_idx(s, lvl):
    sp = jnp.zeros((_PP[lvl], _SP), jnp.int32).at[:_P[lvl]].set(s)
    return sp.reshape(-1)


def kernel(x, s0, s1, s2, s3, D0, D1, D2, D3, U0, U1, U2, U3,
           We0, be0, We1, be1, We2, be2, We3, be3,
           Wfe, bfe, Wfd, bfd,
           Wd0, bd0, Wd1, bd1, Wd2, bd2, Wd3, bd3):
    S = [s0, s1, s2, s3]
    D = [D0, D1, D2, D3]
    U = [U0, U1, U2, U3]
    We = [We0, We1, We2, We3]
    be = [be0, be1, be2, be3]
    Wd = [Wd0, Wd1, Wd2, Wd3]
    bd = [bd0, bd1, bd2, bd3]

    idx = [_pad_idx(S[i], i) for i in range(4)]

    h = x.transpose(1, 0, 2).reshape(_P[0], _B * _FE[0])
    h = jnp.pad(h, ((0, 0), (0, 128 - _B * _FE[0])))
    for i in range(4):
        g = _sc_gather(h, idx[i])
        g3 = g.reshape(_PP[i], _SP, h.shape[1])
        wblk = _blockdiag(We[i], _FE[i], _FE[i + 1])
        if i == 0:
            wblk = jnp.pad(wblk, ((0, 0), (0, 128 - _B * _FE[0]), (0, 0)))
        t = _conv(g3, wblk, _packed_bias(be[i]), act=True)
        h = _pool(D[i], t, _P[i] - 1)

    h4 = h.reshape(_P[4], _B, _FE[4]).transpose(1, 0, 2).reshape(_B, _P[4] * _FE[4])
    y5 = _fc(h4, Wfe, bfe[None, :], Wfd, bfd[None, :])
    y = y5.reshape(_B, _P[4], _FD[0]).transpose(1, 0, 2).reshape(_P[4], _B * _FD[0])

    for i in range(4):
        lvl = 3 - i
        limit = _P[lvl + 1] if i == 0 else _P[lvl + 1] - 1
        y = _pool(U[lvl], y, limit)
        g = _sc_gather(y, idx[lvl])
        g3 = g.reshape(_PP[lvl], _SP, y.shape[1])
        wblk = _blockdiag(Wd[i], _FD[i], _FD[i + 1])
        final = i == 3
        y = _conv(g3, wblk, _packed_bias(bd[i]), act=not final,
                  final_mask_limit=_P[0] - 1 if final else None)

    out = y[:_P[0]].reshape(_P[0], _B, _FD[4]).transpose(1, 0, 2)
    return out
